# baseline (speedup 1.0000x reference)
import jax
import jax.numpy as jnp
import numpy as np
from jax import lax
from jax.experimental import pallas as pl
from jax.experimental.pallas import tpu as pltpu
from jax.experimental.pallas import tpu_sc as plsc

N_NODES = 10000
NP = 10240
D_FEAT = 128
C = 16
E = 320000
CH = 128
NCH = 2560
RPS = NP // 16
K = 8

HOP_SPLIT = (96, 64)
DEG_SPLIT = (104, 56)

FR = N_NODES * C // 128
FP = NP * C // 128


def _make_deg_body(cnt0, cnt1):
    cmax = max(cnt0, cnt1)
    nr = cmax // K

    def body(e_hbm, ones_hbm, zeros_hbm, d_hbm, dst_v, obuf, acc, ssem):
        c = lax.axis_index("c")
        s = lax.axis_index("s")
        base = jnp.where(c == 0, s * cnt0, 16 * cnt0 + s * cnt1)
        nrc = jnp.where(c == 0, cnt0 // K, cnt1 // K)
        d1 = pltpu.async_copy(e_hbm.at[pl.ds(base, cmax)], dst_v, ssem)
        d2 = pltpu.async_copy(ones_hbm, obuf, ssem)
        d3 = pltpu.async_copy(zeros_hbm, acc.at[pl.ds(s * RPS, RPS)], ssem)
        d1.wait()
        d2.wait()
        d3.wait()
        plsc.subcore_barrier()

        def rnd(r, carry):
            @pl.when(r < nrc)
            def _():
                for m in range(K):
                    pltpu.async_copy(obuf, acc.at[dst_v.at[r * K + m]],
                                     ssem, add=True)

            @pl.when((r >= 1) & (r <= nrc))
            def _():
                for m in range(K):
                    pltpu.make_async_copy(ones_hbm, obuf, ssem).wait()
            return carry

        lax.fori_loop(0, nr, rnd, 0)

        @pl.when(nrc == nr)
        def _():
            for m in range(K):
                pltpu.make_async_copy(ones_hbm, obuf, ssem).wait()
        plsc.subcore_barrier()
        pltpu.sync_copy(acc.at[pl.ds(s * RPS, RPS)],
                        d_hbm.at[c, pl.ds(s * RPS, RPS)])

    return body


def _make_hop_body(cnt0, cnt1):
    cmax = max(cnt0, cnt1)
    nr = cmax // K

    def body(t_hbm, e_hbm, zeros_hbm, p_hbm, src_v, dst_v, gbuf, acc, tspm,
             gsem, ssem):
        c = lax.axis_index("c")
        s = lax.axis_index("s")
        base = jnp.where(c == 0, s * cnt0, 16 * cnt0 + s * cnt1)
        nrc = jnp.where(c == 0, cnt0 // K, cnt1 // K)
        d1 = pltpu.async_copy(e_hbm.at[0, pl.ds(base, cmax)], src_v, gsem)
        d2 = pltpu.async_copy(e_hbm.at[1, pl.ds(base, cmax)], dst_v, gsem)
        d3 = pltpu.async_copy(zeros_hbm, acc.at[pl.ds(s * RPS, RPS)], ssem)
        d4 = pltpu.async_copy(t_hbm.at[pl.ds(s * RPS, RPS)],
                              tspm.at[pl.ds(s * RPS, RPS)], ssem)
        d1.wait()
        d2.wait()
        d3.wait()
        d4.wait()
        plsc.subcore_barrier()

        for m in range(K):
            pltpu.async_copy(tspm.at[src_v.at[m]], gbuf.at[m], gsem)

        def rnd(r, carry):
            off = lax.rem(r, 2) * K
            noff = K - off

            @pl.when((r >= 1) & (r <= nrc))
            def _():
                for m in range(K):
                    pltpu.make_async_copy(zeros_hbm.at[pl.ds(0, CH)],
                                          gbuf.at[0], ssem).wait()

            @pl.when(r + 1 < nrc)
            def _():
                for m in range(K):
                    pltpu.async_copy(tspm.at[src_v.at[(r + 1) * K + m]],
                                     gbuf.at[noff + m], gsem)

            @pl.when(r < nrc)
            def _():
                for m in range(K):
                    pltpu.make_async_copy(zeros_hbm.at[pl.ds(0, CH)],
                                          gbuf.at[0], gsem).wait()
                    pltpu.async_copy(gbuf.at[off + m],
                                     acc.at[dst_v.at[r * K + m]],
                                     ssem, add=True)
            return carry

        lax.fori_loop(0, nr, rnd, 0)

        @pl.when(nrc == nr)
        def _():
            for m in range(K):
                pltpu.make_async_copy(zeros_hbm.at[pl.ds(0, CH)],
                                      gbuf.at[0], ssem).wait()
        plsc.subcore_barrier()
        pltpu.sync_copy(acc.at[pl.ds(s * RPS, RPS)],
                        p_hbm.at[c, pl.ds(s * RPS, RPS)])

    return body


_SC_PARAMS = pltpu.CompilerParams(use_tc_tiling_on_sc=False)


def _sc_deg(edges, ones_in, zeros_in):
    mesh = plsc.VectorSubcoreMesh(core_axis_name="c", subcore_axis_name="s")
    cmax = max(DEG_SPLIT)
    fn = pl.kernel(
        _make_deg_body(*DEG_SPLIT),
        out_type=jax.ShapeDtypeStruct((2, NP, C), jnp.float32),
        mesh=mesh,
        compiler_params=_SC_PARAMS,
        scratch_types=[
            pltpu.VMEM((cmax, CH), jnp.int32),
            pltpu.VMEM((CH, C), jnp.float32),
            pltpu.VMEM_SHARED((NP, C), jnp.float32),
            pltpu.SemaphoreType.DMA,
        ],
    )
    return fn(edges, ones_in, zeros_in)


def _sc_hop(t, edges, zeros_in):
    mesh = plsc.VectorSubcoreMesh(core_axis_name="c", subcore_axis_name="s")
    cmax = max(HOP_SPLIT)
    fn = pl.kernel(
        _make_hop_body(*HOP_SPLIT),
        out_type=jax.ShapeDtypeStruct((2, NP, C), jnp.float32),
        mesh=mesh,
        compiler_params=_SC_PARAMS,
        scratch_types=[
            pltpu.VMEM((cmax, CH), jnp.int32),
            pltpu.VMEM((cmax, CH), jnp.int32),
            pltpu.VMEM((2 * K, CH, C), jnp.float32),
            pltpu.VMEM_SHARED((NP, C), jnp.float32),
            pltpu.VMEM_SHARED((NP, C), jnp.float32),
            pltpu.SemaphoreType.DMA,
            pltpu.SemaphoreType.DMA,
        ],
    )
    return fn(t, edges, zeros_in)


def _tc_mm_kbody(xp_ref, wb_ref, z_ref):
    z_ref[...] = lax.dot_general(xp_ref[...], wb_ref[...],
                                 (((1,), (0,)), ((), ())),
                                 preferred_element_type=jnp.float32)


def _tc_mm(xp, wb):
    return pl.pallas_call(
        _tc_mm_kbody,
        out_shape=jax.ShapeDtypeStruct((FR, 128), jnp.float32),
    )(xp, wb)


def _tc1_kbody(z_ref, d_ref, t1_ref, dv_ref):
    deg = d_ref[0, :FR, :] + d_ref[1, :FR, :] + 1.0
    dv = lax.rsqrt(deg)
    zero_tail = jnp.zeros((FP - FR, 128), jnp.float32)
    dv_ref[:FR, :] = dv
    dv_ref[FR:, :] = zero_tail
    t1_ref[:FR, :] = dv * z_ref[...]
    t1_ref[FR:, :] = zero_tail


def _tc1(z, dpacked):
    return pl.pallas_call(
        _tc1_kbody,
        out_shape=[
            jax.ShapeDtypeStruct((FP, 128), jnp.float32),
            jax.ShapeDtypeStruct((FP, 128), jnp.float32),
        ],
    )(z, dpacked)


def _tc2_kbody(p_ref, t1_ref, dv_ref, o_ref):
    dv = dv_ref[...]
    o_ref[...] = dv * dv * (p_ref[0] + p_ref[1] + t1_ref[...])


def _tc2(p, t1p, dvp):
    return pl.pallas_call(
        _tc2_kbody,
        out_shape=jax.ShapeDtypeStruct((FP, 128), jnp.float32),
    )(p, t1p, dvp)


def _tc3_kbody(q_ref, t3_ref, dv_ref, b_ref, o_ref):
    o_ref[...] = (dv_ref[:FR, :]
                  * (q_ref[0, :FR, :] + q_ref[1, :FR, :] + t3_ref[:FR, :])
                  + b_ref[...])


def _tc3(q, t3p, dvp, bt):
    return pl.pallas_call(
        _tc3_kbody,
        out_shape=jax.ShapeDtypeStruct((FR, 128), jnp.float32),
    )(q, t3p, dvp, bt)


def kernel(x, edge_index, W, b):
    ei32 = edge_index.astype(jnp.int32)
    pad1 = jnp.full((NCH * CH - E,), N_NODES, jnp.int32)
    dst_pad = jnp.concatenate([ei32[1], pad1]).reshape(NCH, CH)
    pad = jnp.full((2, NCH * CH - E), N_NODES, jnp.int32)
    edges = jnp.concatenate([ei32, pad], axis=1).reshape(2, NCH, CH)
    ones_in = np.ones((CH, C), np.float32)
    zeros_in = np.zeros((RPS, C), np.float32)

    xp = x.reshape(FR, 8 * D_FEAT)
    wb = jnp.kron(jnp.eye(8, dtype=jnp.float32), W.T)
    bt = jnp.tile(b, 8).reshape(1, 128)

    dparts = _sc_deg(dst_pad, ones_in, zeros_in)
    zf = _tc_mm(xp, wb)
    t1f, dvf = _tc1(zf, dparts.reshape(2, FP, 128))
    p = _sc_hop(t1f.reshape(NP, C), edges, zeros_in)
    t3f = _tc2(p.reshape(2, FP, 128), t1f, dvf)
    q = _sc_hop(t3f.reshape(NP, C), edges, zeros_in)
    outf = _tc3(q.reshape(2, FP, 128), t3f, dvf, bt)
    return outf.reshape(N_NODES, C)


# speedup vs baseline: 1.0868x; 1.0868x over previous
import jax
import jax.numpy as jnp
import numpy as np
from jax import lax
from jax.experimental import pallas as pl
from jax.experimental.pallas import tpu as pltpu
from jax.experimental.pallas import tpu_sc as plsc

N_NODES = 10000
NP = 10240
D_FEAT = 128
C = 16
E = 320000
CH = 128
NCH = 2560
RPS = NP // 16
K = 8

HOP_SPLIT = (88, 72)
DEG_SPLIT = (104, 56)

FR = N_NODES * C // 128
FP = NP * C // 128


def _make_deg_body(cnt0, cnt1):
    cmax = max(cnt0, cnt1)
    nr = cmax // K

    def body(e_hbm, ones_hbm, zeros_hbm, d_hbm, dst_v, obuf, acc, ssem):
        c = lax.axis_index("c")
        s = lax.axis_index("s")
        base = jnp.where(c == 0, s * cnt0, 16 * cnt0 + s * cnt1)
        nrc = jnp.where(c == 0, cnt0 // K, cnt1 // K)
        d1 = pltpu.async_copy(e_hbm.at[1, pl.ds(base, cmax)], dst_v, ssem)
        d2 = pltpu.async_copy(ones_hbm, obuf, ssem)
        d3 = pltpu.async_copy(zeros_hbm, acc.at[pl.ds(s * RPS, RPS)], ssem)
        d1.wait()
        d2.wait()
        d3.wait()
        plsc.subcore_barrier()

        def rnd(r, carry):
            @pl.when(r < nrc)
            def _():
                for m in range(K):
                    pltpu.async_copy(obuf, acc.at[dst_v.at[r * K + m]],
                                     ssem, add=True)

            @pl.when((r >= 1) & (r <= nrc))
            def _():
                for m in range(K):
                    pltpu.make_async_copy(ones_hbm, obuf, ssem).wait()
            return carry

        lax.fori_loop(0, nr, rnd, 0)

        @pl.when(nrc == nr)
        def _():
            for m in range(K):
                pltpu.make_async_copy(ones_hbm, obuf, ssem).wait()
        plsc.subcore_barrier()
        pltpu.sync_copy(acc.at[pl.ds(s * RPS, RPS)],
                        d_hbm.at[c, pl.ds(s * RPS, RPS)])

    return body


def _make_hop_body(cnt0, cnt1):
    cmax = max(cnt0, cnt1)
    nr = cmax // K

    def body(t_hbm, e_hbm, zeros_hbm, p_hbm, src_v, dst_v, gbuf, acc, tspm,
             gsem, ssem):
        c = lax.axis_index("c")
        s = lax.axis_index("s")
        base = jnp.where(c == 0, s * cnt0, 16 * cnt0 + s * cnt1)
        nrc = jnp.where(c == 0, cnt0 // K, cnt1 // K)
        d1 = pltpu.async_copy(e_hbm.at[0, pl.ds(base, cmax)], src_v, gsem)
        d2 = pltpu.async_copy(e_hbm.at[1, pl.ds(base, cmax)], dst_v, gsem)
        d3 = pltpu.async_copy(zeros_hbm, acc.at[pl.ds(s * RPS, RPS)], ssem)
        d4 = pltpu.async_copy(t_hbm.at[pl.ds(s * RPS, RPS)],
                              tspm.at[pl.ds(s * RPS, RPS)], ssem)
        d1.wait()
        d2.wait()
        d3.wait()
        d4.wait()
        plsc.subcore_barrier()

        for m in range(K):
            pltpu.async_copy(tspm.at[src_v.at[m]], gbuf.at[m], gsem)

        def rnd(r, carry):
            off = lax.rem(r, 2) * K
            noff = K - off

            @pl.when((r >= 1) & (r <= nrc))
            def _():
                for m in range(K):
                    pltpu.make_async_copy(zeros_hbm.at[pl.ds(0, CH)],
                                          gbuf.at[0], ssem).wait()

            @pl.when(r + 1 < nrc)
            def _():
                for m in range(K):
                    pltpu.async_copy(tspm.at[src_v.at[(r + 1) * K + m]],
                                     gbuf.at[noff + m], gsem)

            @pl.when(r < nrc)
            def _():
                for m in range(K):
                    pltpu.make_async_copy(zeros_hbm.at[pl.ds(0, CH)],
                                          gbuf.at[0], gsem).wait()
                    pltpu.async_copy(gbuf.at[off + m],
                                     acc.at[dst_v.at[r * K + m]],
                                     ssem, add=True)
            return carry

        lax.fori_loop(0, nr, rnd, 0)

        @pl.when(nrc == nr)
        def _():
            for m in range(K):
                pltpu.make_async_copy(zeros_hbm.at[pl.ds(0, CH)],
                                      gbuf.at[0], ssem).wait()
        plsc.subcore_barrier()
        pltpu.sync_copy(acc.at[pl.ds(s * RPS, RPS)],
                        p_hbm.at[c, pl.ds(s * RPS, RPS)])

    return body


_SC_PARAMS = pltpu.CompilerParams(use_tc_tiling_on_sc=False)


def _sc_deg(edges, ones_in, zeros_in):
    mesh = plsc.VectorSubcoreMesh(core_axis_name="c", subcore_axis_name="s")
    cmax = max(DEG_SPLIT)
    fn = pl.kernel(
        _make_deg_body(*DEG_SPLIT),
        out_type=jax.ShapeDtypeStruct((2, NP, C), jnp.float32),
        mesh=mesh,
        compiler_params=_SC_PARAMS,
        scratch_types=[
            pltpu.VMEM((cmax, CH), jnp.int32),
            pltpu.VMEM((CH, C), jnp.float32),
            pltpu.VMEM_SHARED((NP, C), jnp.float32),
            pltpu.SemaphoreType.DMA,
        ],
    )
    return fn(edges, ones_in, zeros_in)


def _sc_hop(t, edges, zeros_in):
    mesh = plsc.VectorSubcoreMesh(core_axis_name="c", subcore_axis_name="s")
    cmax = max(HOP_SPLIT)
    fn = pl.kernel(
        _make_hop_body(*HOP_SPLIT),
        out_type=jax.ShapeDtypeStruct((2, NP, C), jnp.float32),
        mesh=mesh,
        compiler_params=_SC_PARAMS,
        scratch_types=[
            pltpu.VMEM((cmax, CH), jnp.int32),
            pltpu.VMEM((cmax, CH), jnp.int32),
            pltpu.VMEM((2 * K, CH, C), jnp.float32),
            pltpu.VMEM_SHARED((NP, C), jnp.float32),
            pltpu.VMEM_SHARED((NP, C), jnp.float32),
            pltpu.SemaphoreType.DMA,
            pltpu.SemaphoreType.DMA,
        ],
    )
    return fn(t, edges, zeros_in)


def _tc_mm_kbody(xp_ref, wb_ref, z_ref):
    z_ref[...] = lax.dot_general(xp_ref[...], wb_ref[...],
                                 (((1,), (0,)), ((), ())),
                                 preferred_element_type=jnp.float32)


def _tc_mm(xp, wb):
    return pl.pallas_call(
        _tc_mm_kbody,
        out_shape=jax.ShapeDtypeStruct((FR, 128), jnp.float32),
    )(xp, wb)


def _tc1_kbody(z_ref, d_ref, t1_ref, dv_ref):
    deg = d_ref[0, :FR, :] + d_ref[1, :FR, :] + 1.0
    dv = lax.rsqrt(deg)
    zero_tail = jnp.zeros((FP - FR, 128), jnp.float32)
    dv_ref[:FR, :] = dv
    dv_ref[FR:, :] = zero_tail
    t1_ref[:FR, :] = dv * z_ref[...]
    t1_ref[FR:, :] = zero_tail


def _tc1(z, dpacked):
    return pl.pallas_call(
        _tc1_kbody,
        out_shape=[
            jax.ShapeDtypeStruct((FP, 128), jnp.float32),
            jax.ShapeDtypeStruct((FP, 128), jnp.float32),
        ],
    )(z, dpacked)


def _tc2_kbody(p_ref, t1_ref, dv_ref, o_ref):
    dv = dv_ref[...]
    o_ref[...] = dv * dv * (p_ref[0] + p_ref[1] + t1_ref[...])


def _tc2(p, t1p, dvp):
    return pl.pallas_call(
        _tc2_kbody,
        out_shape=jax.ShapeDtypeStruct((FP, 128), jnp.float32),
    )(p, t1p, dvp)


def _tc3_kbody(q_ref, t3_ref, dv_ref, b_ref, o_ref):
    o_ref[...] = (dv_ref[:FR, :]
                  * (q_ref[0, :FR, :] + q_ref[1, :FR, :] + t3_ref[:FR, :])
                  + b_ref[...])


def _tc3(q, t3p, dvp, bt):
    return pl.pallas_call(
        _tc3_kbody,
        out_shape=jax.ShapeDtypeStruct((FR, 128), jnp.float32),
    )(q, t3p, dvp, bt)


def kernel(x, edge_index, W, b):
    ei32 = edge_index.astype(jnp.int32)
    pad = jnp.full((2, NCH * CH - E), N_NODES, jnp.int32)
    edges = jnp.concatenate([ei32, pad], axis=1).reshape(2, NCH, CH)
    ones_in = np.ones((CH, C), np.float32)
    zeros_in = np.zeros((RPS, C), np.float32)

    xp = x.reshape(FR, 8 * D_FEAT)
    wb = jnp.kron(jnp.eye(8, dtype=jnp.float32), W.T)
    bt = jnp.tile(b, 8).reshape(1, 128)

    dparts = _sc_deg(edges, ones_in, zeros_in)
    zf = _tc_mm(xp, wb)
    t1f, dvf = _tc1(zf, dparts.reshape(2, FP, 128))
    p = _sc_hop(t1f.reshape(NP, C), edges, zeros_in)
    t3f = _tc2(p.reshape(2, FP, 128), t1f, dvf)
    q = _sc_hop(t3f.reshape(NP, C), edges, zeros_in)
    outf = _tc3(q.reshape(2, FP, 128), t3f, dvf, bt)
    return outf.reshape(N_NODES, C)
